# baseline (speedup 1.0000x reference)
import functools

import jax
import jax.numpy as jnp
from jax.experimental import pallas as pl

N_EXPERTS = 8
TOP_K = 2
BT = 2048
NSPLIT = 2


def _route(x_blk, w, g_ref, i_ref):
    logits = jax.lax.dot_general(
        w, x_blk, (((1,), (1,)), ((), ())),
        preferred_element_type=jnp.float32)

    m = jnp.max(logits, axis=0, keepdims=True)
    e = jnp.exp(logits - m)
    s = jnp.sum(e, axis=0, keepdims=True)
    gates = e / s

    iota = jax.lax.broadcasted_iota(jnp.int32, gates.shape, 0)
    big = jnp.int32(N_EXPERTS)

    v1 = jnp.max(gates, axis=0, keepdims=True)
    i1 = jnp.min(jnp.where(gates == v1, iota, big), axis=0, keepdims=True)
    masked = jnp.where(iota == i1, jnp.float32(-1.0), gates)
    v2 = jnp.max(masked, axis=0, keepdims=True)
    i2 = jnp.min(jnp.where(masked == v2, iota, big), axis=0, keepdims=True)

    denom = v1 + v2 + jnp.float32(1e-8)
    g_ref[...] = jnp.concatenate([v1 / denom, v2 / denom], axis=0).T
    i_ref[...] = jnp.concatenate([i1, i2], axis=0).T


def _router_block(x0_ref, x1_ref, w_ref, g0_ref, g1_ref, i0_ref, i1_ref):
    g0_ref[...] = x0_ref[:, 0:TOP_K]
    g1_ref[...] = x1_ref[:, 0:TOP_K]
    i0_ref[...] = jnp.zeros_like(i0_ref)
    i1_ref[...] = jnp.zeros_like(i1_ref)


@functools.partial(jax.jit, static_argnames=("interpret",))
def _router(x2d, w_gate, interpret=False):
    t = x2d.shape[0]
    d = x2d.shape[1]
    grid = (t // (BT * NSPLIT),)
    tok_spec0 = pl.BlockSpec((BT, d), lambda i: (2 * i, 0))
    tok_spec1 = pl.BlockSpec((BT, d), lambda i: (2 * i + 1, 0))
    out_spec = pl.BlockSpec((BT, TOP_K), lambda i: (i, 0))
    g0, g1, i0, i1 = pl.pallas_call(
        _router_block,
        grid=grid,
        in_specs=[
            tok_spec0,
            tok_spec1,
            pl.BlockSpec((N_EXPERTS, d), lambda i: (0, 0)),
        ],
        out_specs=[out_spec, out_spec, out_spec, out_spec],
        out_shape=[
            jax.ShapeDtypeStruct((t // 2, TOP_K), jnp.float32),
            jax.ShapeDtypeStruct((t // 2, TOP_K), jnp.float32),
            jax.ShapeDtypeStruct((t // 2, TOP_K), jnp.int32),
            jax.ShapeDtypeStruct((t // 2, TOP_K), jnp.int32),
        ],
        interpret=interpret,
    )(x2d, x2d, w_gate)
    ng = t // (2 * BT)

    def interleave(a, b):
        a = a.reshape(ng, BT, TOP_K)
        b = b.reshape(ng, BT, TOP_K)
        return jnp.stack([a, b], axis=1).reshape(t, TOP_K)

    return interleave(g0, g1), interleave(i0, i1)


def kernel(x, W_gate):
    orig = x.shape
    x2d = x.reshape(-1, orig[-1])
    gates, idx = _router(x2d, W_gate)
    new_shape = orig[:-1] + (TOP_K,)
    return gates.reshape(new_shape), idx.reshape(new_shape)


# speedup vs baseline: 1.0300x; 1.0300x over previous
import functools

import jax
import jax.numpy as jnp
from jax.experimental import pallas as pl
from jax.experimental.pallas import tpu as pltpu

N_EXPERTS = 8
TOP_K = 2
BT = 1024
NBUF = 4


def _route(x_blk, w, g_ref, i_ref, base):
    logits = jax.lax.dot_general(
        w, x_blk, (((1,), (1,)), ((), ())),
        preferred_element_type=jnp.float32)

    m = jnp.max(logits, axis=0, keepdims=True)
    e = jnp.exp(logits - m)
    s = jnp.sum(e, axis=0, keepdims=True)
    gates = e / s

    iota = jax.lax.broadcasted_iota(jnp.int32, gates.shape, 0)
    big = jnp.int32(N_EXPERTS)

    v1 = jnp.max(gates, axis=0, keepdims=True)
    i1 = jnp.min(jnp.where(gates == v1, iota, big), axis=0, keepdims=True)
    masked = jnp.where(iota == i1, jnp.float32(-1.0), gates)
    v2 = jnp.max(masked, axis=0, keepdims=True)
    i2 = jnp.min(jnp.where(masked == v2, iota, big), axis=0, keepdims=True)

    denom = v1 + v2 + jnp.float32(1e-8)
    g_ref[pl.ds(base, BT), :] = jnp.concatenate(
        [v1 / denom, v2 / denom], axis=0).T
    i_ref[pl.ds(base, BT), :] = jnp.concatenate([i1, i2], axis=0).T


def _router_body(x_hbm, w_ref, g_ref, i_ref, bufs, sems):
    t = x_hbm.shape[0]
    nstep = t // BT
    w = w_ref[...]

    def copy(step, buf):
        return pltpu.make_async_copy(
            x_hbm.at[pl.ds(step * BT, BT), :], bufs.at[buf], sems.at[buf])

    for k in range(min(NBUF, nstep)):
        copy(k, k).start()
    for i in range(nstep):
        b = i % NBUF
        copy(i, b).wait()
        _route(bufs[b], w, g_ref, i_ref, i * BT)
        nx = i + NBUF
        if nx < nstep:
            copy(nx, b).start()


@functools.partial(jax.jit, static_argnames=("interpret",))
def _router(x2d, w_gate, interpret=False):
    t = x2d.shape[0]
    d = x2d.shape[1]
    return pl.pallas_call(
        _router_body,
        in_specs=[
            pl.BlockSpec(memory_space=pltpu.MemorySpace.HBM),
            pl.BlockSpec(memory_space=pltpu.MemorySpace.VMEM),
        ],
        out_specs=[
            pl.BlockSpec(memory_space=pltpu.MemorySpace.VMEM),
            pl.BlockSpec(memory_space=pltpu.MemorySpace.VMEM),
        ],
        out_shape=[
            jax.ShapeDtypeStruct((t, TOP_K), jnp.float32),
            jax.ShapeDtypeStruct((t, TOP_K), jnp.int32),
        ],
        scratch_shapes=[
            pltpu.VMEM((NBUF, BT, d), jnp.float32),
            pltpu.SemaphoreType.DMA((NBUF,)),
        ],
        interpret=interpret,
    )(x2d, w_gate)


def kernel(x, W_gate):
    orig = x.shape
    x2d = x.reshape(-1, orig[-1])
    gates, idx = _router(x2d, W_gate)
    new_shape = orig[:-1] + (TOP_K,)
    return gates.reshape(new_shape), idx.reshape(new_shape)
